# baseline (speedup 1.0000x reference)
import functools

import jax
import jax.numpy as jnp
from jax import lax
from jax.experimental import pallas as pl
from jax.experimental.pallas import tpu as pltpu
from jax.experimental.pallas import tpu_sc as plsc

_N = 10000
_PN = 10240
_C = 128
_E = 320000
_NW = 32
_CH = 128
_NCH = 80
_EPW = _CH * _NCH
_EPAD = _NW * _EPW
_TOTCH = _EPAD // _CH
_NSUB = 160
_NE0 = 144
_NE1 = 16
_NC0 = 112
_NC1 = 48
_RPT = _PN // 16
_BLK = 1024
_NBLK = _PN // _BLK
_NB = 2

_mesh = plsc.VectorSubcoreMesh(core_axis_name="c", subcore_axis_name="s")


def _deg_pipe(dst_hbm, s, lo, n, deg_sp, idx_v, ones_v, sem):
    pltpu.sync_copy(dst_hbm.at[s, pl.ds(lo, n)], idx_v.at[pl.ds(0, n)])

    def fire(i, _):
        pltpu.async_copy(ones_v, deg_sp.at[idx_v.at[i]], sem, add=True)
        return ()

    def drain(i, _):
        pltpu.make_async_copy(ones_v, deg_sp.at[idx_v.at[i]], sem).wait()
        return ()

    lax.fori_loop(0, n, fire, ())
    lax.fori_loop(0, n, drain, ())


def _deg_body(dst_hbm, deg_out, deg_sp, idx_v, ones_v, sem):
    c = lax.axis_index("c")
    s = lax.axis_index("s")
    zero16 = jnp.zeros((16,), jnp.float32)
    for j in range(_CH // 16):
        ones_v[pl.ds(j * 16, 16)] = zero16
    for p in range(_RPT // _CH):
        pltpu.sync_copy(ones_v, deg_sp.at[pl.ds(s * _RPT + p * _CH, _CH)])
    one16 = jnp.ones((16,), jnp.float32)
    for j in range(_CH // 16):
        ones_v[pl.ds(j * 16, 16)] = one16
    plsc.subcore_barrier()

    @pl.when(c == 0)
    def _():
        _deg_pipe(dst_hbm, s, 0, _NC0, deg_sp, idx_v, ones_v, sem)

    @pl.when(c == 1)
    def _():
        _deg_pipe(dst_hbm, s, _NC0, _NC1, deg_sp, idx_v, ones_v, sem)

    plsc.subcore_barrier()
    pltpu.sync_copy(deg_sp.at[pl.ds(s * _RPT, _RPT)],
                    deg_out.at[c, pl.ds(s * _RPT, _RPT)])


_deg_call = pl.kernel(
    _deg_body,
    out_type=jax.ShapeDtypeStruct((2, _PN), jnp.float32),
    mesh=_mesh,
    scratch_types=[
        pltpu.VMEM_SHARED((_PN,), jnp.float32),
        pltpu.VMEM((_NC0, _CH), jnp.int32),
        pltpu.VMEM((_CH,), jnp.float32),
        pltpu.SemaphoreType.DMA,
    ],
)


def _edge_pipe(src_hbm, dst_hbm, s, lo, n, z_hbm, dinv_hbm, g_sp, u_sp,
               srcb_v, dstb_v, rows_v, val_v,
               sem_i, sem_g, sem_s, sem_d, sem_t):
    _NI = 3

    def i_start(g):
        pltpu.async_copy(src_hbm.at[s, lo + g], srcb_v.at[g % _NI], sem_i)
        pltpu.async_copy(dst_hbm.at[s, lo + g], dstb_v.at[g % _NI], sem_i)

    def i_wait(g):
        pltpu.make_async_copy(src_hbm.at[s, lo + g], srcb_v.at[g % _NI],
                              sem_i).wait()
        pltpu.make_async_copy(dst_hbm.at[s, lo + g], dstb_v.at[g % _NI],
                              sem_i).wait()

    def du_start(g):
        pltpu.async_copy(dinv_hbm.at[dstb_v.at[g % _NI]], val_v.at[g % _NB],
                         sem_d)

    def du_wait(g):
        pltpu.make_async_copy(dinv_hbm.at[dstb_v.at[g % _NI]],
                              val_v.at[g % _NB], sem_d).wait()

    def su_start(g):
        pltpu.async_copy(val_v.at[g % _NB], u_sp.at[srcb_v.at[g % _NI]],
                         sem_t, add=True)

    def su_wait(g):
        pltpu.make_async_copy(val_v.at[g % _NB], u_sp.at[srcb_v.at[g % _NI]],
                              sem_t).wait()

    def gv_start(g):
        pltpu.async_copy(z_hbm.at[srcb_v.at[g % _NI]], rows_v.at[g % _NB],
                         sem_g)

    def gv_wait(g):
        pltpu.make_async_copy(z_hbm.at[srcb_v.at[g % _NI]],
                              rows_v.at[g % _NB], sem_g).wait()

    def sv_start(g):
        pltpu.async_copy(rows_v.at[g % _NB], g_sp.at[dstb_v.at[g % _NI]],
                         sem_s, add=True)

    def sv_wait(g):
        pltpu.make_async_copy(rows_v.at[g % _NB], g_sp.at[dstb_v.at[g % _NI]],
                              sem_s).wait()

    i_start(0)
    i_wait(0)
    gv_start(0)
    du_start(0)
    i_start(1)

    def ring(g, _):
        gv_wait(g)
        sv_start(g)
        du_wait(g)
        su_start(g)

        @pl.when(g >= 1)
        def _():
            sv_wait(g - 1)
            su_wait(g - 1)

        @pl.when(g + 2 < n)
        def _():
            i_start(g + 2)

        @pl.when(g + 1 < n)
        def _():
            i_wait(g + 1)
            gv_start(g + 1)
            du_start(g + 1)

        return ()

    lax.fori_loop(0, n, ring, ())
    sv_wait(n - 1)
    su_wait(n - 1)


def _edge_body(src_hbm, dst_hbm, z_hbm, dinv_hbm, g_out, u_out, g_sp, u_sp,
               srcb_v, dstb_v, rows_v, val_v,
               sem_i, sem_g, sem_s, sem_d, sem_t):
    c = lax.axis_index("c")
    s = lax.axis_index("s")
    zero16 = jnp.zeros((16,), jnp.float32)

    def zrow_loop(r, _):
        for j in range(_C // 16):
            rows_v[0, r, pl.ds(j * 16, 16)] = zero16
        return ()

    lax.fori_loop(0, _CH, zrow_loop, ())
    for j in range(_CH // 16):
        val_v[0, pl.ds(j * 16, 16)] = zero16

    for p in range(_RPT // _CH):
        pltpu.sync_copy(rows_v.at[0],
                        g_sp.at[pl.ds(s * _RPT + p * _CH, _CH)])
        pltpu.sync_copy(val_v.at[0],
                        u_sp.at[pl.ds(s * _RPT + p * _CH, _CH)])

    plsc.subcore_barrier()

    @pl.when(c == 0)
    def _():
        _edge_pipe(src_hbm, dst_hbm, s, 0, _NE0, z_hbm, dinv_hbm, g_sp, u_sp,
                   srcb_v, dstb_v, rows_v, val_v,
                   sem_i, sem_g, sem_s, sem_d, sem_t)

    @pl.when(c == 1)
    def _():
        _edge_pipe(src_hbm, dst_hbm, s, _NE0, _NE1, z_hbm, dinv_hbm, g_sp,
                   u_sp, srcb_v, dstb_v, rows_v, val_v,
                   sem_i, sem_g, sem_s, sem_d, sem_t)

    plsc.subcore_barrier()
    pltpu.sync_copy(g_sp.at[pl.ds(s * _RPT, _RPT)],
                    g_out.at[c, pl.ds(s * _RPT, _RPT)])
    pltpu.sync_copy(u_sp.at[pl.ds(s * _RPT, _RPT)],
                    u_out.at[c, pl.ds(s * _RPT, _RPT)])


_edge_call = pl.kernel(
    _edge_body,
    out_type=(jax.ShapeDtypeStruct((2, _PN, _C), jnp.float32),
              jax.ShapeDtypeStruct((2, _PN), jnp.float32)),
    mesh=_mesh,
    scratch_types=[
        pltpu.VMEM_SHARED((_PN, _C), jnp.float32),
        pltpu.VMEM_SHARED((_PN,), jnp.float32),
        pltpu.VMEM((3, _CH), jnp.int32),
        pltpu.VMEM((3, _CH), jnp.int32),
        pltpu.VMEM((_NB, _CH, _C), jnp.float32),
        pltpu.VMEM((_NB, _CH), jnp.float32),
        pltpu.SemaphoreType.DMA,
        pltpu.SemaphoreType.DMA,
        pltpu.SemaphoreType.DMA,
        pltpu.SemaphoreType.DMA,
        pltpu.SemaphoreType.DMA,
    ],
)


def _stats_body(x_ref, gamma_ref, out_ref):
    i = pl.program_id(0)

    @pl.when(i == 0)
    def _():
        out_ref[...] = jnp.zeros_like(out_ref)

    xb = x_ref[...]
    out_ref[0:1, :] += jnp.sum(xb, axis=0, keepdims=True)
    out_ref[1:2, :] += jnp.sum(xb * xb, axis=0, keepdims=True)

    @pl.when(i == _NBLK - 1)
    def _():
        mean = out_ref[0:1, :] / _N
        var = out_ref[1:2, :] / _N - mean * mean
        out_ref[0:1, :] = mean
        out_ref[1:2, :] = gamma_ref[...] * lax.rsqrt(var + 1e-5)


def _stats_call(x_pad, gamma):
    return pl.pallas_call(
        _stats_body,
        grid=(_NBLK,),
        in_specs=[pl.BlockSpec((_BLK, _C), lambda i: (i, 0)),
                  pl.BlockSpec((1, _C), lambda i: (0, 0))],
        out_specs=pl.BlockSpec((8, _C), lambda i: (0, 0)),
        out_shape=jax.ShapeDtypeStruct((8, _C), jnp.float32),
    )(x_pad, gamma)


def _z_body(x_ref, stats_ref, beta_ref, d0_ref, d1_ref, z_ref, dinv_ref):
    i = pl.program_id(0)
    mean = stats_ref[0:1, :]
    scale = stats_ref[1:2, :]
    dsum = d0_ref[...] + d1_ref[...] + 1.0
    dv = lax.rsqrt(dsum)
    rows = i * _BLK + lax.broadcasted_iota(jnp.int32, (_BLK, 1), 0)
    dv = jnp.where(rows < _N, dv, 0.0)
    bn = (x_ref[...] - mean) * scale + beta_ref[...]
    z_ref[...] = dv * bn
    dinv_ref[...] = dv


def _z_call(x_pad, stats, beta, d0, d1):
    return pl.pallas_call(
        _z_body,
        grid=(_NBLK,),
        in_specs=[pl.BlockSpec((_BLK, _C), lambda i: (i, 0)),
                  pl.BlockSpec((8, _C), lambda i: (0, 0)),
                  pl.BlockSpec((1, _C), lambda i: (0, 0)),
                  pl.BlockSpec((_BLK, 1), lambda i: (i, 0)),
                  pl.BlockSpec((_BLK, 1), lambda i: (i, 0))],
        out_specs=[pl.BlockSpec((_BLK, _C), lambda i: (i, 0)),
                   pl.BlockSpec((_BLK, 1), lambda i: (i, 0))],
        out_shape=(jax.ShapeDtypeStruct((_PN, _C), jnp.float32),
                   jax.ShapeDtypeStruct((_PN, 1), jnp.float32)),
    )(x_pad, stats, beta, d0, d1)


def _final_body(g0_ref, g1_ref, z_ref, dinv_ref, u0_ref, u1_ref,
                w1_ref, b1_ref, w2_ref, b2_ref, out_ref, acc_ref):
    i = pl.program_id(0)
    dv = dinv_ref[...]
    p = dv * (g0_ref[0] + g1_ref[0] + z_ref[...])
    pre = jnp.dot(p, w1_ref[...], preferred_element_type=jnp.float32)
    pre = pre + b1_ref[...]
    leaky = jnp.where(pre >= 0, pre, 0.1 * pre)
    sv = dv * (u0_ref[...] + u1_ref[...] + dv)
    part = jnp.sum(sv * leaky, axis=0, keepdims=True)

    @pl.when(i == 0)
    def _():
        acc_ref[...] = jnp.zeros_like(acc_ref)

    acc_ref[0:1, :] += part

    @pl.when(i == _NBLK - 1)
    def _():
        pooled = acc_ref[0:1, :] / _N
        out_ref[...] = (jnp.dot(pooled, w2_ref[...],
                                preferred_element_type=jnp.float32)
                        + b2_ref[...])


def _final_call(g_p, z, dinv, u0, u1, W1, b1, W2, b2):
    return pl.pallas_call(
        _final_body,
        grid=(_NBLK,),
        in_specs=[pl.BlockSpec((1, _BLK, _C), lambda i: (0, i, 0)),
                  pl.BlockSpec((1, _BLK, _C), lambda i: (1, i, 0)),
                  pl.BlockSpec((_BLK, _C), lambda i: (i, 0)),
                  pl.BlockSpec((_BLK, 1), lambda i: (i, 0)),
                  pl.BlockSpec((_BLK, 1), lambda i: (i, 0)),
                  pl.BlockSpec((_BLK, 1), lambda i: (i, 0)),
                  pl.BlockSpec((_C, _C), lambda i: (0, 0)),
                  pl.BlockSpec((1, _C), lambda i: (0, 0)),
                  pl.BlockSpec((_C, _C), lambda i: (0, 0)),
                  pl.BlockSpec((1, _C), lambda i: (0, 0))],
        out_specs=pl.BlockSpec((1, _C), lambda i: (0, 0)),
        out_shape=jax.ShapeDtypeStruct((1, _C), jnp.float32),
        scratch_shapes=[pltpu.VMEM((8, _C), jnp.float32)],
    )(g_p, g_p, z, dinv, u0, u1, W1, b1, W2, b2)


def kernel(x, edge_index, bn_gamma, bn_beta, W1, b1, W2, b2):
    src = edge_index[0].astype(jnp.int32)
    dst = edge_index[1].astype(jnp.int32)
    pad_cfg = (0, _EPAD - _E)
    src_p = jnp.pad(src, pad_cfg, constant_values=_PN - 1).reshape(16, _NSUB, _CH)
    dst_p = jnp.pad(dst, pad_cfg, constant_values=_PN - 1).reshape(16, _NSUB, _CH)
    x_pad = jnp.pad(x, ((0, _PN - _N), (0, 0)))

    deg_p = _deg_call(dst_p)
    stats = _stats_call(x_pad, bn_gamma.reshape(1, _C))
    d0 = deg_p[0].reshape(_PN, 1)
    d1 = deg_p[1].reshape(_PN, 1)
    z, dinv = _z_call(x_pad, stats, bn_beta.reshape(1, _C), d0, d1)

    g_p, u_p = _edge_call(src_p, dst_p, z, dinv.reshape(_PN))

    out = _final_call(g_p, z, dinv,
                      u_p[0].reshape(_PN, 1), u_p[1].reshape(_PN, 1),
                      W1, b1.reshape(1, _C), W2, b2.reshape(1, _C))
    return out


# speedup vs baseline: 1.2345x; 1.2345x over previous
import functools

import jax
import jax.numpy as jnp
from jax import lax
from jax.experimental import pallas as pl
from jax.experimental.pallas import tpu as pltpu
from jax.experimental.pallas import tpu_sc as plsc

_N = 10000
_PN = 10240
_C = 128
_E = 320000
_NW = 32
_CH = 80
_NCH = 128
_EPW = _CH * _NCH
_EPAD = _NW * _EPW
_TOTCH = _EPAD // _CH
_NSUB = 256
_NE0 = 180
_NE1 = 76
_NC0 = 176
_NC1 = 80
_RPT = _PN // 16
_BLK = 1024
_NBLK = _PN // _BLK
_NB = 2

_mesh = plsc.VectorSubcoreMesh(core_axis_name="c", subcore_axis_name="s")


def _deg_pipe(dst_hbm, s, lo, n, deg_sp, idx_v, ones_v, sem):
    pltpu.sync_copy(dst_hbm.at[s, pl.ds(lo, n)], idx_v.at[pl.ds(0, n)])

    def fire(i, _):
        pltpu.async_copy(ones_v, deg_sp.at[idx_v.at[i]], sem, add=True)
        return ()

    def drain(i, _):
        pltpu.make_async_copy(ones_v, deg_sp.at[idx_v.at[i]], sem).wait()
        return ()

    lax.fori_loop(0, n, fire, ())
    lax.fori_loop(0, n, drain, ())


def _deg_body(dst_hbm, deg_out, deg_sp, idx_v, ones_v, sem):
    c = lax.axis_index("c")
    s = lax.axis_index("s")
    zero16 = jnp.zeros((16,), jnp.float32)
    for j in range(_CH // 16):
        ones_v[pl.ds(j * 16, 16)] = zero16
    for p in range(_RPT // _CH):
        pltpu.sync_copy(ones_v, deg_sp.at[pl.ds(s * _RPT + p * _CH, _CH)])
    one16 = jnp.ones((16,), jnp.float32)
    for j in range(_CH // 16):
        ones_v[pl.ds(j * 16, 16)] = one16
    plsc.subcore_barrier()

    @pl.when(c == 0)
    def _():
        _deg_pipe(dst_hbm, s, 0, _NC0, deg_sp, idx_v, ones_v, sem)

    @pl.when(c == 1)
    def _():
        _deg_pipe(dst_hbm, s, _NC0, _NC1, deg_sp, idx_v, ones_v, sem)

    plsc.subcore_barrier()
    pltpu.sync_copy(deg_sp.at[pl.ds(s * _RPT, _RPT)],
                    deg_out.at[c, pl.ds(s * _RPT, _RPT)])


_deg_call = pl.kernel(
    _deg_body,
    out_type=jax.ShapeDtypeStruct((2, _PN), jnp.float32),
    mesh=_mesh,
    scratch_types=[
        pltpu.VMEM_SHARED((_PN,), jnp.float32),
        pltpu.VMEM((_NC0, _CH), jnp.int32),
        pltpu.VMEM((_CH,), jnp.float32),
        pltpu.SemaphoreType.DMA,
    ],
)


def _edge_pipe(src_hbm, dst_hbm, s, lo, n, z_hbm, dinv_hbm, g_sp, u_sp,
               srcb_v, dstb_v, rows_v, val_v,
               sem_i, sem_g, sem_s, sem_d, sem_t):
    _NI = 3

    def i_start(g):
        pltpu.async_copy(src_hbm.at[s, lo + g], srcb_v.at[g % _NI], sem_i)
        pltpu.async_copy(dst_hbm.at[s, lo + g], dstb_v.at[g % _NI], sem_i)

    def i_wait(g):
        pltpu.make_async_copy(src_hbm.at[s, lo + g], srcb_v.at[g % _NI],
                              sem_i).wait()
        pltpu.make_async_copy(dst_hbm.at[s, lo + g], dstb_v.at[g % _NI],
                              sem_i).wait()

    def du_start(g):
        pltpu.async_copy(dinv_hbm.at[dstb_v.at[g % _NI]], val_v.at[g % _NB],
                         sem_d)

    def du_wait(g):
        pltpu.make_async_copy(dinv_hbm.at[dstb_v.at[g % _NI]],
                              val_v.at[g % _NB], sem_d).wait()

    def su_start(g):
        pltpu.async_copy(val_v.at[g % _NB], u_sp.at[srcb_v.at[g % _NI]],
                         sem_t, add=True)

    def su_wait(g):
        pltpu.make_async_copy(val_v.at[g % _NB], u_sp.at[srcb_v.at[g % _NI]],
                              sem_t).wait()

    def gv_start(g):
        pltpu.async_copy(z_hbm.at[srcb_v.at[g % _NI]], rows_v.at[g % _NB],
                         sem_g)

    def gv_wait(g):
        pltpu.make_async_copy(z_hbm.at[srcb_v.at[g % _NI]],
                              rows_v.at[g % _NB], sem_g).wait()

    def sv_start(g):
        pltpu.async_copy(rows_v.at[g % _NB], g_sp.at[dstb_v.at[g % _NI]],
                         sem_s, add=True)

    def sv_wait(g):
        pltpu.make_async_copy(rows_v.at[g % _NB], g_sp.at[dstb_v.at[g % _NI]],
                              sem_s).wait()

    i_start(0)
    i_wait(0)
    gv_start(0)
    du_start(0)
    i_start(1)

    def ring(g, _):
        gv_wait(g)
        sv_start(g)
        du_wait(g)
        su_start(g)

        @pl.when(g >= 1)
        def _():
            sv_wait(g - 1)
            su_wait(g - 1)

        @pl.when(g + 2 < n)
        def _():
            i_start(g + 2)

        @pl.when(g + 1 < n)
        def _():
            i_wait(g + 1)
            gv_start(g + 1)
            du_start(g + 1)

        return ()

    lax.fori_loop(0, n, ring, ())
    sv_wait(n - 1)
    su_wait(n - 1)


def _edge_body(src_hbm, dst_hbm, z_hbm, dinv_hbm, g_out, u_out, g_sp, u_sp,
               srcb_v, dstb_v, rows_v, val_v,
               sem_i, sem_g, sem_s, sem_d, sem_t):
    c = lax.axis_index("c")
    s = lax.axis_index("s")
    zero16 = jnp.zeros((16,), jnp.float32)

    def zrow_loop(r, _):
        for j in range(_C // 16):
            rows_v[0, r, pl.ds(j * 16, 16)] = zero16
        return ()

    lax.fori_loop(0, _CH, zrow_loop, ())
    for j in range(_CH // 16):
        val_v[0, pl.ds(j * 16, 16)] = zero16

    for p in range(_RPT // _CH):
        pltpu.sync_copy(rows_v.at[0],
                        g_sp.at[pl.ds(s * _RPT + p * _CH, _CH)])
        pltpu.sync_copy(val_v.at[0],
                        u_sp.at[pl.ds(s * _RPT + p * _CH, _CH)])

    plsc.subcore_barrier()

    @pl.when(c == 0)
    def _():
        _edge_pipe(src_hbm, dst_hbm, s, 0, _NE0, z_hbm, dinv_hbm, g_sp, u_sp,
                   srcb_v, dstb_v, rows_v, val_v,
                   sem_i, sem_g, sem_s, sem_d, sem_t)

    @pl.when(c == 1)
    def _():
        _edge_pipe(src_hbm, dst_hbm, s, _NE0, _NE1, z_hbm, dinv_hbm, g_sp,
                   u_sp, srcb_v, dstb_v, rows_v, val_v,
                   sem_i, sem_g, sem_s, sem_d, sem_t)

    plsc.subcore_barrier()
    pltpu.sync_copy(g_sp.at[pl.ds(s * _RPT, _RPT)],
                    g_out.at[c, pl.ds(s * _RPT, _RPT)])
    pltpu.sync_copy(u_sp.at[pl.ds(s * _RPT, _RPT)],
                    u_out.at[c, pl.ds(s * _RPT, _RPT)])


_edge_call = pl.kernel(
    _edge_body,
    out_type=(jax.ShapeDtypeStruct((2, _PN, _C), jnp.float32),
              jax.ShapeDtypeStruct((2, _PN), jnp.float32)),
    mesh=_mesh,
    scratch_types=[
        pltpu.VMEM_SHARED((_PN, _C), jnp.float32),
        pltpu.VMEM_SHARED((_PN,), jnp.float32),
        pltpu.VMEM((3, _CH), jnp.int32),
        pltpu.VMEM((3, _CH), jnp.int32),
        pltpu.VMEM((_NB, _CH, _C), jnp.float32),
        pltpu.VMEM((_NB, _CH), jnp.float32),
        pltpu.SemaphoreType.DMA,
        pltpu.SemaphoreType.DMA,
        pltpu.SemaphoreType.DMA,
        pltpu.SemaphoreType.DMA,
        pltpu.SemaphoreType.DMA,
    ],
)


def _stats_body(x_ref, gamma_ref, out_ref):
    i = pl.program_id(0)

    @pl.when(i == 0)
    def _():
        out_ref[...] = jnp.zeros_like(out_ref)

    xb = x_ref[...]
    out_ref[0:1, :] += jnp.sum(xb, axis=0, keepdims=True)
    out_ref[1:2, :] += jnp.sum(xb * xb, axis=0, keepdims=True)

    @pl.when(i == _NBLK - 1)
    def _():
        mean = out_ref[0:1, :] / _N
        var = out_ref[1:2, :] / _N - mean * mean
        out_ref[0:1, :] = mean
        out_ref[1:2, :] = gamma_ref[...] * lax.rsqrt(var + 1e-5)


def _stats_call(x_pad, gamma):
    return pl.pallas_call(
        _stats_body,
        grid=(_NBLK,),
        in_specs=[pl.BlockSpec((_BLK, _C), lambda i: (i, 0)),
                  pl.BlockSpec((1, _C), lambda i: (0, 0))],
        out_specs=pl.BlockSpec((8, _C), lambda i: (0, 0)),
        out_shape=jax.ShapeDtypeStruct((8, _C), jnp.float32),
    )(x_pad, gamma)


def _z_body(x_ref, stats_ref, beta_ref, d0_ref, d1_ref, z_ref, dinv_ref):
    i = pl.program_id(0)
    mean = stats_ref[0:1, :]
    scale = stats_ref[1:2, :]
    dsum = d0_ref[...] + d1_ref[...] + 1.0
    dv = lax.rsqrt(dsum)
    rows = i * _BLK + lax.broadcasted_iota(jnp.int32, (_BLK, 1), 0)
    dv = jnp.where(rows < _N, dv, 0.0)
    bn = (x_ref[...] - mean) * scale + beta_ref[...]
    z_ref[...] = dv * bn
    dinv_ref[...] = dv


def _z_call(x_pad, stats, beta, d0, d1):
    return pl.pallas_call(
        _z_body,
        grid=(_NBLK,),
        in_specs=[pl.BlockSpec((_BLK, _C), lambda i: (i, 0)),
                  pl.BlockSpec((8, _C), lambda i: (0, 0)),
                  pl.BlockSpec((1, _C), lambda i: (0, 0)),
                  pl.BlockSpec((_BLK, 1), lambda i: (i, 0)),
                  pl.BlockSpec((_BLK, 1), lambda i: (i, 0))],
        out_specs=[pl.BlockSpec((_BLK, _C), lambda i: (i, 0)),
                   pl.BlockSpec((_BLK, 1), lambda i: (i, 0))],
        out_shape=(jax.ShapeDtypeStruct((_PN, _C), jnp.float32),
                   jax.ShapeDtypeStruct((_PN, 1), jnp.float32)),
    )(x_pad, stats, beta, d0, d1)


def _final_body(g0_ref, g1_ref, z_ref, dinv_ref, u0_ref, u1_ref,
                w1_ref, b1_ref, w2_ref, b2_ref, out_ref, acc_ref):
    i = pl.program_id(0)
    dv = dinv_ref[...]
    p = dv * (g0_ref[0] + g1_ref[0] + z_ref[...])
    pre = jnp.dot(p, w1_ref[...], preferred_element_type=jnp.float32)
    pre = pre + b1_ref[...]
    leaky = jnp.where(pre >= 0, pre, 0.1 * pre)
    sv = dv * (u0_ref[...] + u1_ref[...] + dv)
    part = jnp.sum(sv * leaky, axis=0, keepdims=True)

    @pl.when(i == 0)
    def _():
        acc_ref[...] = jnp.zeros_like(acc_ref)

    acc_ref[0:1, :] += part

    @pl.when(i == _NBLK - 1)
    def _():
        pooled = acc_ref[0:1, :] / _N
        out_ref[...] = (jnp.dot(pooled, w2_ref[...],
                                preferred_element_type=jnp.float32)
                        + b2_ref[...])


def _final_call(g_p, z, dinv, u0, u1, W1, b1, W2, b2):
    return pl.pallas_call(
        _final_body,
        grid=(_NBLK,),
        in_specs=[pl.BlockSpec((1, _BLK, _C), lambda i: (0, i, 0)),
                  pl.BlockSpec((1, _BLK, _C), lambda i: (1, i, 0)),
                  pl.BlockSpec((_BLK, _C), lambda i: (i, 0)),
                  pl.BlockSpec((_BLK, 1), lambda i: (i, 0)),
                  pl.BlockSpec((_BLK, 1), lambda i: (i, 0)),
                  pl.BlockSpec((_BLK, 1), lambda i: (i, 0)),
                  pl.BlockSpec((_C, _C), lambda i: (0, 0)),
                  pl.BlockSpec((1, _C), lambda i: (0, 0)),
                  pl.BlockSpec((_C, _C), lambda i: (0, 0)),
                  pl.BlockSpec((1, _C), lambda i: (0, 0))],
        out_specs=pl.BlockSpec((1, _C), lambda i: (0, 0)),
        out_shape=jax.ShapeDtypeStruct((1, _C), jnp.float32),
        scratch_shapes=[pltpu.VMEM((8, _C), jnp.float32)],
    )(g_p, g_p, z, dinv, u0, u1, W1, b1, W2, b2)


def kernel(x, edge_index, bn_gamma, bn_beta, W1, b1, W2, b2):
    src = edge_index[0].astype(jnp.int32)
    dst = edge_index[1].astype(jnp.int32)
    pad_cfg = (0, _EPAD - _E)
    src_p = jnp.pad(src, pad_cfg, constant_values=_PN - 1).reshape(16, _NSUB, _CH)
    dst_p = jnp.pad(dst, pad_cfg, constant_values=_PN - 1).reshape(16, _NSUB, _CH)
    x_pad = jnp.pad(x, ((0, _PN - _N), (0, 0)))

    deg_p = _deg_call(dst_p)
    stats = _stats_call(x_pad, bn_gamma.reshape(1, _C))
    d0 = deg_p[0].reshape(_PN, 1)
    d1 = deg_p[1].reshape(_PN, 1)
    z, dinv = _z_call(x_pad, stats, bn_beta.reshape(1, _C), d0, d1)

    g_p, u_p = _edge_call(src_p, dst_p, z, dinv.reshape(_PN))

    out = _final_call(g_p, z, dinv,
                      u_p[0].reshape(_PN, 1), u_p[1].reshape(_PN, 1),
                      W1, b1.reshape(1, _C), W2, b2.reshape(1, _C))
    return out
